# Initial kernel scaffold; baseline (speedup 1.0000x reference)
import jax
import jax.numpy as jnp
from jax.experimental import pallas as pl


def kernel(x, emb_table, W, b, gamma, beta):
    raise NotImplementedError("write your pallas kernel here")


# speedup vs baseline: 1.5274x; 1.5274x over previous
import jax
import jax.numpy as jnp
from jax.experimental import pallas as pl

N, C, L, D_PIX, D_MODEL = 16, 32, 128, 64, 128
T = 64


def _fused_kernel(x_ref, w_ref, be_ref, g_ref, bt_ref, o_ref):
    xb = x_ref[...].reshape(T * L, D_PIX)
    y = jax.lax.dot_general(
        xb, w_ref[...], (((1,), (1,)), ((), ())),
        preferred_element_type=jnp.float32,
    )
    y = (y.reshape(T, L, D_MODEL) + be_ref[...][None, :, :]).reshape(T * L, D_MODEL)
    mean = jnp.mean(y, axis=-1, keepdims=True)
    yc = y - mean
    var = jnp.mean(yc * yc, axis=-1, keepdims=True)
    out = yc * jax.lax.rsqrt(var + 1e-5) * g_ref[...] + bt_ref[...]
    o_ref[...] = out.reshape(T, L, D_MODEL)


def kernel(x, emb_table, W, b, gamma, beta):
    xr = x.reshape(N * C, L, D_PIX)
    emb_b = emb_table + b[None, :]
    out = pl.pallas_call(
        _fused_kernel,
        grid=(N * C // T,),
        in_specs=[
            pl.BlockSpec((T, L, D_PIX), lambda i: (i, 0, 0)),
            pl.BlockSpec((D_MODEL, D_PIX), lambda i: (0, 0)),
            pl.BlockSpec((L, D_MODEL), lambda i: (0, 0)),
            pl.BlockSpec((1, D_MODEL), lambda i: (0, 0)),
            pl.BlockSpec((1, D_MODEL), lambda i: (0, 0)),
        ],
        out_specs=pl.BlockSpec((T, L, D_MODEL), lambda i: (i, 0, 0)),
        out_shape=jax.ShapeDtypeStruct((N * C, L, D_MODEL), jnp.float32),
    )(xr, W, emb_b, gamma.reshape(1, D_MODEL), beta.reshape(1, D_MODEL))
    return out.reshape(N, C, L, D_MODEL)
